# baseline (speedup 1.0000x reference)
import jax
import jax.numpy as jnp
from jax import lax
from jax.experimental import pallas as pl
from jax.experimental.pallas import tpu as pltpu
from jax.experimental.pallas import tpu_sc as plsc

BATCH = 16384
EMB = 32
NROWS = 1000000
_NC = 2
_NS = 16
_NW = _NC * _NS
_BPW = BATCH // _NW
_NCH = NROWS // 128
_TROWS = NROWS // 4
_CHUNK = 128


def _mesh():
    return plsc.VectorSubcoreMesh(core_axis_name="c", subcore_axis_name="s")


def _transpose_block(vin, vout, nrows):
    f_lo = lax.iota(jnp.int32, 16)
    f_hi = f_lo + 16
    for i in range(nrows):
        row = jnp.full((16,), i, jnp.int32)
        for s in range(8):
            col = jnp.full((16,), 4 * i + s // 2, jnp.int32)
            fvec = f_hi if s % 2 else f_lo
            v = plsc.load_gather(vin, [fvec, col])
            plsc.store_scatter(vout, [row, f_lo + s * 16], v)


def _relayout_body(utabT, itabT, utail, itail, utmp, itmp,
                   vin_u, vout_u, vin_i, vout_i, vtail,
                   ru, wu, ri, wi):
    w = lax.axis_index("s") * _NC + lax.axis_index("c")
    base = _NCH // _NW
    rem = _NCH - base * _NW
    start = w * base + jnp.minimum(w, rem)
    end = start + base + jnp.where(w < rem, 1, 0)

    def rd(tab, dst, sem, c):
        off = pl.multiple_of(c * 128, 128)
        pltpu.async_copy(tab.at[:, pl.ds(off, 128)],
                         dst.at[:, pl.ds(0, 128)], sem)

    def wr(src, tmp, sem, c):
        off = pl.multiple_of(c * 32, 8)
        pltpu.async_copy(src, tmp.at[pl.ds(off, 32), :], sem)

    def rd_wait(tab, dst, sem):
        pltpu.make_async_copy(tab.at[:, pl.ds(0, 128)],
                              dst.at[:, pl.ds(0, 128)], sem).wait()

    def wr_wait(src, tmp, sem):
        pltpu.make_async_copy(src, tmp.at[pl.ds(0, 32), :], sem).wait()

    rd(utabT, vin_u, ru, start)
    rd(itabT, vin_i, ri, start)

    def step(c, carry):
        rd_wait(utabT, vin_u, ru)

        @pl.when(c > start)
        def _():
            wr_wait(vout_u, utmp, wu)
        _transpose_block(vin_u, vout_u, 32)
        wr(vout_u, utmp, wu, c)

        @pl.when(c + 1 < end)
        def _():
            rd(utabT, vin_u, ru, c + 1)

        rd_wait(itabT, vin_i, ri)

        @pl.when(c > start)
        def _():
            wr_wait(vout_i, itmp, wi)
        _transpose_block(vin_i, vout_i, 32)
        wr(vout_i, itmp, wi, c)

        @pl.when(c + 1 < end)
        def _():
            rd(itabT, vin_i, ri, c + 1)
        return carry

    lax.fori_loop(start, end, step, 0)
    wr_wait(vout_u, utmp, wu)
    wr_wait(vout_i, itmp, wi)

    @pl.when(w == _NW - 1)
    def _():
        f_lo = lax.iota(jnp.int32, 16)
        for tail, tmp in ((utail, utmp), (itail, itmp)):
            pltpu.sync_copy(tail, vtail)
            for i in range(16):
                for s in range(8):
                    src = (4 * i + s // 2) * 32 + 16 * (s % 2)
                    idx = f_lo + src
                    plsc.store_scatter(
                        vout_u,
                        [jnp.full((16,), i, jnp.int32),
                         f_lo + s * 16],
                        plsc.load_gather(vtail, [idx]))
            pltpu.sync_copy(vout_u.at[pl.ds(0, 16), :],
                            tmp.at[pl.ds(_NCH * 32, 16), :])


def _relayout_call(utabT, itabT, utail, itail):
    f = pl.kernel(
        _relayout_body, mesh=_mesh(),
        out_type=(
            jax.ShapeDtypeStruct((_TROWS, 128), jnp.float32),
            jax.ShapeDtypeStruct((_TROWS, 128), jnp.float32),
        ),
        scratch_types=[
            pltpu.VMEM((EMB, 129), jnp.float32),
            pltpu.VMEM((EMB, 128), jnp.float32),
            pltpu.VMEM((EMB, 129), jnp.float32),
            pltpu.VMEM((EMB, 128), jnp.float32),
            pltpu.VMEM((64 * EMB,), jnp.float32),
            pltpu.SemaphoreType.DMA,
            pltpu.SemaphoreType.DMA,
            pltpu.SemaphoreType.DMA,
            pltpu.SemaphoreType.DMA,
        ],
        compiler_params=pltpu.CompilerParams(needs_layout_passes=False),
    )
    return f(utabT, itabT, utail, itail)


def _gather_body(uids_hbm, iids_hbm, utmp, itmp, uout, iout,
                 ids_v, ridx, rows, outv_u, outv_i, sem):
    w = lax.axis_index("s") * _NC + lax.axis_index("c")
    ri16 = lax.iota(jnp.int32, 16)

    ids_off = pl.multiple_of(w * _BPW, 8)
    for ids_hbm, tmp, outv in ((uids_hbm, utmp, outv_u),
                               (iids_hbm, itmp, outv_i)):
        pltpu.sync_copy(ids_hbm.at[pl.ds(ids_off, _BPW)], ids_v)

        def mk_ridx(g, c):
            ridx[pl.ds(g * 16, 16)] = ids_v[pl.ds(g * 16, 16)] >> 2
            return c
        lax.fori_loop(0, _BPW // 16, mk_ridx, 0)

        waits = []
        for j in range(_BPW // _CHUNK):
            waits.append(pltpu.async_copy(
                tmp.at[ridx.at[pl.ds(j * _CHUNK, _CHUNK)]],
                rows.at[pl.ds(j * _CHUNK, _CHUNK)], sem))
        for wt in waits:
            wt.wait()

        def extract(g, c):
            u = ids_v[pl.ds(g * 16, 16)]
            kvec = ri16 + g * 16
            cbase = (u & 3) * 32
            for f in range(EMB):
                outv[f, pl.ds(g * 16, 16)] = plsc.load_gather(
                    rows, [kvec, cbase + f])
            return c
        lax.fori_loop(0, _BPW // 16, extract, 0)

    pltpu.sync_copy(outv_u, uout.at[:, pl.ds(w * _BPW, _BPW)])
    pltpu.sync_copy(outv_i, iout.at[:, pl.ds(w * _BPW, _BPW)])


def _gather_call(uids, iids, utmp, itmp):
    f = pl.kernel(
        _gather_body, mesh=_mesh(),
        out_type=(
            jax.ShapeDtypeStruct((EMB, BATCH), jnp.float32),
            jax.ShapeDtypeStruct((EMB, BATCH), jnp.float32),
        ),
        scratch_types=[
            pltpu.VMEM((_BPW,), jnp.int32),
            pltpu.VMEM((_BPW,), jnp.int32),
            pltpu.VMEM((_BPW, 128), jnp.float32),
            pltpu.VMEM((EMB, _BPW), jnp.float32),
            pltpu.VMEM((EMB, _BPW), jnp.float32),
            pltpu.SemaphoreType.DMA,
        ],
        compiler_params=pltpu.CompilerParams(needs_layout_passes=False),
    )
    return f(uids, iids, utmp, itmp)


def kernel(user_ids, item_ids, user_table, item_table):
    uids = user_ids.astype(jnp.int32)
    iids = item_ids.astype(jnp.int32)
    utmp, itmp = _relayout_call(
        user_table.T, item_table.T,
        user_table[_NCH * 128:].reshape(-1), item_table[_NCH * 128:].reshape(-1))
    uoutT, ioutT = _gather_call(uids, iids, utmp, itmp)
    return uoutT.T, ioutT.T


# speedup vs baseline: 2.2671x; 2.2671x over previous
import jax
import jax.numpy as jnp
from jax import lax
from jax.experimental import pallas as pl
from jax.experimental.pallas import tpu as pltpu
from jax.experimental.pallas import tpu_sc as plsc

BATCH = 16384
EMB = 32
_NC = 2
_NS = 16
_NW = _NC * _NS
_BPW = BATCH // _NW
_CHUNK = 128
_NCHUNK = _BPW // _CHUNK


def _emb_body(uids_hbm, iids_hbm, utab_hbm, itab_hbm,
              uout_hbm, iout_hbm,
              uidx, iidx, urows, irows, sem):
    wid = lax.axis_index("s") * _NC + lax.axis_index("c")
    pltpu.sync_copy(uids_hbm.at[wid], uidx)
    pltpu.sync_copy(iids_hbm.at[wid], iidx)
    waits = []
    for j in range(_NCHUNK):
        waits.append(pltpu.async_copy(
            utab_hbm.at[uidx.at[j]],
            urows.at[pl.ds(j * _CHUNK, _CHUNK)], sem))
        waits.append(pltpu.async_copy(
            itab_hbm.at[iidx.at[j]],
            irows.at[pl.ds(j * _CHUNK, _CHUNK)], sem))
    for w in waits:
        w.wait()
    pltpu.sync_copy(urows, uout_hbm.at[wid])
    pltpu.sync_copy(irows, iout_hbm.at[wid])


def _emb_call(uids, iids, user_table, item_table):
    mesh = plsc.VectorSubcoreMesh(core_axis_name="c", subcore_axis_name="s")
    f = pl.kernel(
        _emb_body, mesh=mesh,
        out_type=(
            jax.ShapeDtypeStruct((_NW, _BPW, EMB), jnp.float32),
            jax.ShapeDtypeStruct((_NW, _BPW, EMB), jnp.float32),
        ),
        scratch_types=[
            pltpu.VMEM((_NCHUNK, _CHUNK), jnp.int32),
            pltpu.VMEM((_NCHUNK, _CHUNK), jnp.int32),
            pltpu.VMEM((_BPW, EMB), jnp.float32),
            pltpu.VMEM((_BPW, EMB), jnp.float32),
            pltpu.SemaphoreType.DMA,
        ],
        compiler_params=pltpu.CompilerParams(use_tc_tiling_on_sc=False),
    )
    return f(uids, iids, user_table, item_table)


def kernel(user_ids, item_ids, user_table, item_table):
    uids = user_ids.astype(jnp.int32).reshape(_NW, _NCHUNK, _CHUNK)
    iids = item_ids.astype(jnp.int32).reshape(_NW, _NCHUNK, _CHUNK)
    uout, iout = _emb_call(uids, iids, user_table, item_table)
    return uout.reshape(BATCH, EMB), iout.reshape(BATCH, EMB)
